# Initial kernel scaffold; baseline (speedup 1.0000x reference)
import jax
import jax.numpy as jnp
from jax.experimental import pallas as pl


def kernel(triples, norm_vector_weight):
    raise NotImplementedError("write your pallas kernel here")


# speedup vs baseline: 6.6953x; 6.6953x over previous
import functools

import jax
import jax.numpy as jnp
from jax import lax
from jax.experimental import pallas as pl
from jax.experimental.pallas import tpu as pltpu
from jax.experimental.pallas import tpu_sc as plsc

EMBED_DIM = 128
GCH = 128


@functools.cache
def _make_gather(num_rows: int, total: int):
    info = plsc.get_sparse_core_info()
    nc, ns = info.num_cores, info.num_subcores
    nw = nc * ns
    assert total % (nw * GCH) == 0
    per_w = total // nw
    n_g = per_w // GCH

    mesh = plsc.VectorSubcoreMesh(core_axis_name="c", subcore_axis_name="s")

    @functools.partial(
        pl.kernel,
        out_type=jax.ShapeDtypeStruct((total, EMBED_DIM), jnp.float32),
        mesh=mesh,
        scratch_types=[
            pltpu.VMEM((per_w,), jnp.int32),
            pltpu.VMEM((GCH, EMBED_DIM), jnp.float32),
            pltpu.SemaphoreType.DMA,
        ],
    )
    def gather(table_hbm, idx_hbm, out_hbm, idx_v, rows_v, sem):
        wid = lax.axis_index("s") * nc + lax.axis_index("c")
        base = wid * per_w
        pltpu.sync_copy(idx_hbm.at[pl.ds(base, per_w)], idx_v)

        def body(g, carry):
            off = g * GCH
            pltpu.async_copy(
                table_hbm.at[idx_v.at[pl.ds(off, GCH)]], rows_v, sem
            ).wait()
            pltpu.sync_copy(rows_v, out_hbm.at[pl.ds(base + off, GCH)])
            return carry

        lax.fori_loop(0, n_g, body, 0)

    return gather


def kernel(triples, norm_vector_weight):
    b, n, _ = triples.shape
    idx = triples[:, :, 3].reshape(-1).astype(jnp.int32)
    out = _make_gather(norm_vector_weight.shape[0], b * n)(
        norm_vector_weight, idx
    )
    return out.reshape(b, n, 1, 1, EMBED_DIM)


# speedup vs baseline: 8.4621x; 1.2639x over previous
import functools

import jax
import jax.numpy as jnp
from jax import lax
from jax.experimental import pallas as pl
from jax.experimental.pallas import tpu as pltpu
from jax.experimental.pallas import tpu_sc as plsc

EMBED_DIM = 128
GCH = 128


@functools.cache
def _make_gather(num_rows: int, total: int):
    info = plsc.get_sparse_core_info()
    nc, ns = info.num_cores, info.num_subcores
    nw = nc * ns
    assert total % (nw * GCH) == 0
    per_w = total // nw
    n_g = per_w // GCH

    mesh = plsc.VectorSubcoreMesh(core_axis_name="c", subcore_axis_name="s")

    assert n_g % 2 == 0

    @functools.partial(
        pl.kernel,
        out_type=jax.ShapeDtypeStruct((total, EMBED_DIM), jnp.float32),
        mesh=mesh,
        scratch_types=[
            pltpu.VMEM((per_w,), jnp.int32),
            pltpu.VMEM((GCH, EMBED_DIM), jnp.float32),
            pltpu.VMEM((GCH, EMBED_DIM), jnp.float32),
            pltpu.SemaphoreType.DMA,
            pltpu.SemaphoreType.DMA,
        ],
    )
    def gather(table_hbm, idx_hbm, out_hbm, idx_v, rows0, rows1, sem0, sem1):
        wid = lax.axis_index("s") * nc + lax.axis_index("c")
        base = wid * per_w
        pltpu.sync_copy(idx_hbm.at[pl.ds(base, per_w)], idx_v)

        def start(g, buf, sem):
            pltpu.async_copy(table_hbm.at[idx_v.at[pl.ds(g * GCH, GCH)]], buf, sem)

        def drain(buf, sem):
            pltpu.make_async_copy(table_hbm.at[pl.ds(0, GCH)], buf, sem).wait()

        def emit(g, buf, sem):
            drain(buf, sem)
            pltpu.sync_copy(buf, out_hbm.at[pl.ds(base + g * GCH, GCH)])

        start(0, rows0, sem0)
        start(1, rows1, sem1)

        def body(i, carry):
            g = 2 * i
            emit(g, rows0, sem0)
            start(g + 2, rows0, sem0)
            emit(g + 1, rows1, sem1)
            start(g + 3, rows1, sem1)
            return carry

        lax.fori_loop(0, n_g // 2 - 1, body, 0)
        emit(n_g - 2, rows0, sem0)
        emit(n_g - 1, rows1, sem1)

    return gather


def kernel(triples, norm_vector_weight):
    b, n, _ = triples.shape
    idx = triples[:, :, 3].reshape(-1).astype(jnp.int32)
    out = _make_gather(norm_vector_weight.shape[0], b * n)(
        norm_vector_weight, idx
    )
    return out.reshape(b, n, 1, 1, EMBED_DIM)


# speedup vs baseline: 8.5573x; 1.0113x over previous
import functools

import jax
import jax.numpy as jnp
from jax import lax
from jax.experimental import pallas as pl
from jax.experimental.pallas import tpu as pltpu
from jax.experimental.pallas import tpu_sc as plsc

EMBED_DIM = 128
GCH = 128
BLK = 256


@functools.cache
def _make_gather(num_rows: int, total: int):
    info = plsc.get_sparse_core_info()
    nc, ns = info.num_cores, info.num_subcores
    nw = nc * ns
    assert total % (nw * BLK) == 0 and BLK % GCH == 0
    per_w = total // nw
    n_b = per_w // BLK
    g_per_b = BLK // GCH
    assert n_b >= 3

    mesh = plsc.VectorSubcoreMesh(core_axis_name="c", subcore_axis_name="s")

    @functools.partial(
        pl.kernel,
        out_type=jax.ShapeDtypeStruct((total, EMBED_DIM), jnp.float32),
        mesh=mesh,
        scratch_types=[
            pltpu.VMEM((per_w,), jnp.int32),
            pltpu.VMEM((BLK, EMBED_DIM), jnp.float32),
            pltpu.VMEM((BLK, EMBED_DIM), jnp.float32),
            pltpu.SemaphoreType.DMA,
            pltpu.SemaphoreType.DMA,
        ],
    )
    def gather(table_hbm, idx_hbm, out_hbm, idx_v, rows0, rows1, sem0, sem1):
        wid = lax.axis_index("s") * nc + lax.axis_index("c")
        base = wid * per_w
        pltpu.sync_copy(idx_hbm.at[pl.ds(base, per_w)], idx_v)

        def start(b, buf, sem):
            for j in range(g_per_b):
                pltpu.async_copy(
                    table_hbm.at[idx_v.at[pl.ds(b * BLK + j * GCH, GCH)]],
                    buf.at[pl.ds(j * GCH, GCH)],
                    sem,
                )

        def drain(buf, sem):
            pltpu.make_async_copy(table_hbm.at[pl.ds(0, BLK)], buf, sem).wait()

        def emit(b, buf, sem):
            drain(buf, sem)
            pltpu.sync_copy(buf, out_hbm.at[pl.ds(base + b * BLK, BLK)])

        bufs = (rows0, rows1)
        sems = (sem0, sem1)
        start(0, rows0, sem0)
        start(1, rows1, sem1)

        n_pairs = (n_b - 2) // 2

        def body(i, carry):
            b = 2 * i
            emit(b, rows0, sem0)
            start(b + 2, rows0, sem0)
            emit(b + 1, rows1, sem1)
            start(b + 3, rows1, sem1)
            return carry

        lax.fori_loop(0, n_pairs, body, 0)
        for b in range(2 * n_pairs, n_b):
            emit(b, bufs[b % 2], sems[b % 2])
            if b + 2 < n_b:
                start(b + 2, bufs[b % 2], sems[b % 2])

    return gather


def kernel(triples, norm_vector_weight):
    b, n, _ = triples.shape
    idx = triples[:, :, 3].reshape(-1).astype(jnp.int32)
    out = _make_gather(norm_vector_weight.shape[0], b * n)(
        norm_vector_weight, idx
    )
    return out.reshape(b, n, 1, 1, EMBED_DIM)
